# baseline (speedup 1.0000x reference)
import functools

import jax
import jax.numpy as jnp
from jax import lax
from jax.experimental import pallas as pl
from jax.experimental.pallas import tpu as pltpu
from jax.experimental.pallas import tpu_sc as plsc

N_NODES = 10000
N_EDGES = 160000
BATCH = 4
NW = 32
NPW = 313
LAST_ROWS = N_NODES - (NW - 1) * NPW
EPW = N_EDGES // NW

_SC_MESH = dict(core_axis_name="c", subcore_axis_name="s")


def _sc_edge_gather_add(A, B, dst, src):
    C = A.shape[1]
    G = 40
    nb = EPW // G
    mesh = plsc.VectorSubcoreMesh(**_SC_MESH)

    @functools.partial(
        pl.kernel,
        out_type=jax.ShapeDtypeStruct((N_EDGES, C), jnp.float32),
        mesh=mesh,
        compiler_params=pltpu.CompilerParams(needs_layout_passes=False),
        scratch_types=[
            pltpu.VMEM((G,), jnp.int32),
            pltpu.VMEM((G,), jnp.int32),
            pltpu.VMEM((G, C), jnp.float32),
            pltpu.VMEM((G, C), jnp.float32),
            pltpu.VMEM((G, C), jnp.float32),
            pltpu.SemaphoreType.DMA,
            pltpu.SemaphoreType.DMA,
        ],
    )
    def k(a_hbm, b_hbm, dst_hbm, src_hbm, out_hbm, di, si, ar, br, hr, s1, s2):
        wid = lax.axis_index("s") * 2 + lax.axis_index("c")
        base0 = wid * EPW

        def body(j, carry):
            base = base0 + j * G
            pltpu.sync_copy(dst_hbm.at[pl.ds(base, G)], di)
            pltpu.sync_copy(src_hbm.at[pl.ds(base, G)], si)
            c1 = pltpu.async_copy(a_hbm.at[di], ar, s1)
            c2 = pltpu.async_copy(b_hbm.at[si], br, s2)
            c1.wait()
            c2.wait()
            for g in range(G):
                def fb(ci, _, g=g):
                    sl = pl.ds(ci * 16, 16)
                    hr[g, sl] = jnp.maximum(ar[g, sl] + br[g, sl], 0.0)
                    return 0
                lax.fori_loop(0, C // 16, fb, 0)
            pltpu.sync_copy(hr, out_hbm.at[pl.ds(base, G)])
            return carry

        lax.fori_loop(0, nb, body, 0)

    return k(A, B, dst, src)


def _sc_scatter_max(m, dst):
    C = m.shape[1]
    S = 2000
    G = 16
    nchunk = N_EDGES // S
    ACC = (NPW + 1) * C
    mesh = plsc.VectorSubcoreMesh(**_SC_MESH)

    @functools.partial(
        pl.kernel,
        out_type=jax.ShapeDtypeStruct((N_NODES * C,), jnp.float32),
        mesh=mesh,
        compiler_params=pltpu.CompilerParams(needs_layout_passes=False),
        scratch_types=[
            pltpu.VMEM((S,), jnp.int32),
            pltpu.VMEM((S + 64,), jnp.int32),
            pltpu.VMEM((S + 64,), jnp.int32),
            pltpu.VMEM((G, C), jnp.float32),
            pltpu.VMEM((ACC,), jnp.float32),
            pltpu.SemaphoreType.DMA,
        ],
    )
    def k(m_hbm, dst_hbm, out_hbm, dstc, ids, lds, rows, acc, sem):
        wid = lax.axis_index("s") * 2 + lax.axis_index("c")
        lo = wid * NPW
        hi = lo + NPW
        dummy_eid = wid * EPW
        lanes = lax.iota(jnp.int32, 16)

        def zb(i, _):
            acc[pl.ds(i * 16, 16)] = jnp.zeros((16,), jnp.float32)
            return 0
        lax.fori_loop(0, ACC // 16, zb, 0)

        def chunk_body(cidx, _):
            ebase = cidx * S
            pltpu.sync_copy(dst_hbm.at[pl.ds(ebase, S)], dstc)

            def scan(i, kk):
                dv = dstc[pl.ds(i * 16, 16)]
                msk = (dv >= lo) & (dv < hi)
                mi = msk.astype(jnp.int32)
                pos = kk + plsc.cumsum(mi) - 1
                eid = lanes + (ebase + i * 16)
                plsc.store_scatter(ids, [pos], eid, mask=msk)
                plsc.store_scatter(lds, [pos], dv - lo, mask=msk)
                return kk + jnp.sum(mi)

            kk = lax.fori_loop(0, S // 16, scan, 0)
            ids[pl.ds(kk, 16)] = jnp.full((16,), dummy_eid, jnp.int32)
            lds[pl.ds(kk, 16)] = jnp.full((16,), NPW, jnp.int32)
            nbat = (kk + G - 1) // G

            def flush(j, _):
                cp = pltpu.async_copy(m_hbm.at[ids.at[pl.ds(j * G, G)]], rows, sem)
                cp.wait()
                ldv = lds[pl.ds(j * G, G)]
                for g in range(G):
                    rowbase = ldv[g] * C

                    def fb(ci, _, g=g, rowbase=rowbase):
                        sl = pl.ds(rowbase + ci * 16, 16)
                        acc[sl] = jnp.maximum(acc[sl], rows[g, pl.ds(ci * 16, 16)])
                        return 0
                    lax.fori_loop(0, C // 16, fb, 0)
                return 0

            lax.fori_loop(0, nbat, flush, 0)
            return 0

        lax.fori_loop(0, nchunk, chunk_body, 0)

        @pl.when(wid < NW - 1)
        def _():
            pltpu.sync_copy(acc.at[pl.ds(0, NPW * C)],
                            out_hbm.at[pl.ds(lo * C, NPW * C)])

        @pl.when(wid == NW - 1)
        def _():
            pltpu.sync_copy(acc.at[pl.ds(0, LAST_ROWS * C)],
                            out_hbm.at[pl.ds(lo * C, LAST_ROWS * C)])

    return k(m, dst).reshape(N_NODES, C)


def _mm(xs, ws, *, koffs=None, bias=None, act=None, sel=None, BR=1000, BN=None):
    R = xs[0].shape[0]
    N = ws[0].shape[1]
    if BN is None:
        BN = min(N, 1024)
    nN = N // BN
    nR = R // BR
    if koffs is None:
        koffs = [0] * len(xs)
    nx = len(xs)
    in_specs = []
    args = []
    for xa, w, ko in zip(xs, ws, koffs):
        Kw = w.shape[0]
        in_specs.append(pl.BlockSpec((BR, Kw), lambda n, r, ko=ko: (r, ko)))
        args.append(xa)
    for w in ws:
        in_specs.append(pl.BlockSpec((w.shape[0], BN), lambda n, r: (0, n)))
        args.append(w)
    has_bias = bias is not None
    if has_bias:
        in_specs.append(pl.BlockSpec((1, BN), lambda n, r: (0, n)))
        args.append(bias)
    has_sel = sel is not None
    if has_sel:
        b2d, Gm = sel
        in_specs.append(pl.BlockSpec((BR, 1), lambda n, r: (r, 0)))
        args.append(b2d)
        in_specs.append(pl.BlockSpec((8, BN), lambda n, r: (0, n)))
        args.append(Gm)

    def body(*refs):
        xr = refs[:nx]
        wr = refs[nx:2 * nx]
        rest = list(refs[2 * nx:-1])
        out_ref = refs[-1]
        acc = None
        for a, b in zip(xr, wr):
            p = jnp.dot(a[...], b[...], preferred_element_type=jnp.float32, precision=lax.Precision.HIGHEST)
            acc = p if acc is None else acc + p
        i = 0
        if has_bias:
            acc = acc + rest[i][...]
            i += 1
        if has_sel:
            bref = rest[i][...]
            gref = rest[i + 1][...]
            gsel = jnp.where(bref == 0, gref[0:1],
                             jnp.where(bref == 1, gref[1:2],
                                       jnp.where(bref == 2, gref[2:3], gref[3:4])))
            acc = acc + gsel
        if act == 'relu':
            acc = jnp.maximum(acc, 0.0)
        elif act == 'sigmoid':
            acc = 1.0 / (1.0 + jnp.exp(-acc))
        elif act == 'l2norm':
            nrm = jnp.sqrt(jnp.sum(acc * acc, axis=1, keepdims=True))
            acc = acc / jnp.maximum(nrm, 1e-6)
        out_ref[...] = acc

    return pl.pallas_call(
        body, grid=(nN, nR), in_specs=in_specs,
        out_specs=pl.BlockSpec((BR, BN), lambda n, r: (r, n)),
        out_shape=jax.ShapeDtypeStruct((R, N), jnp.float32))(*args)


def _t1(x, Wcat, bcat, C):
    R, Cin = x.shape
    BR = 1000
    nR = R // BR

    def body(xr, wr, br, o0, o1, o2, o3):
        acc = jnp.dot(xr[...], wr[...], preferred_element_type=jnp.float32, precision=lax.Precision.HIGHEST) + br[...]
        o0[...] = acc[:, 0:C]
        o1[...] = acc[:, C:2 * C]
        o2[...] = acc[:, 2 * C:3 * C]
        o3[...] = acc[:, 3 * C:4 * C]

    return pl.pallas_call(
        body, grid=(nR,),
        in_specs=[pl.BlockSpec((BR, Cin), lambda r: (r, 0)),
                  pl.BlockSpec((Cin, 4 * C), lambda r: (0, 0)),
                  pl.BlockSpec((1, 4 * C), lambda r: (0, 0))],
        out_specs=[pl.BlockSpec((BR, C), lambda r: (r, 0))] * 4,
        out_shape=[jax.ShapeDtypeStruct((R, C), jnp.float32)] * 4,
    )(x, Wcat, bcat)


def _t4(x1, x2, x3, ws, bias, batch2d):
    R = N_NODES
    BR = 1000
    nR = R // BR
    N = ws[0].shape[1]

    def body(x1r, x2r, x3r, w1r, w2r, w3r, br, batr, o, og):
        r = pl.program_id(0)
        acc = (jnp.dot(x1r[...], w1r[...], preferred_element_type=jnp.float32, precision=lax.Precision.HIGHEST)
               + jnp.dot(x2r[...], w2r[...], preferred_element_type=jnp.float32, precision=lax.Precision.HIGHEST)
               + jnp.dot(x3r[...], w3r[...], preferred_element_type=jnp.float32, precision=lax.Precision.HIGHEST)
               + br[...])
        y = jnp.maximum(acc, 0.0)
        o[...] = y
        prev = jnp.where(r == 0, jnp.zeros((8, N), jnp.float32), og[...])
        b = batr[...]
        rows = []
        for i in range(4):
            xm = jnp.where(b == i, y, 0.0)
            rows.append(jnp.maximum(prev[i:i + 1],
                                    jnp.max(xm, axis=0, keepdims=True)))
        rows.append(prev[4:8])
        og[...] = jnp.concatenate(rows, axis=0)

    return pl.pallas_call(
        body, grid=(nR,),
        in_specs=[pl.BlockSpec((BR, x1.shape[1]), lambda r: (r, 0)),
                  pl.BlockSpec((BR, x2.shape[1]), lambda r: (r, 0)),
                  pl.BlockSpec((BR, x3.shape[1]), lambda r: (r, 0)),
                  pl.BlockSpec((x1.shape[1], N), lambda r: (0, 0)),
                  pl.BlockSpec((x2.shape[1], N), lambda r: (0, 0)),
                  pl.BlockSpec((x3.shape[1], N), lambda r: (0, 0)),
                  pl.BlockSpec((1, N), lambda r: (0, 0)),
                  pl.BlockSpec((BR, 1), lambda r: (r, 0))],
        out_specs=[pl.BlockSpec((BR, N), lambda r: (r, 0)),
                   pl.BlockSpec((8, N), lambda r: (0, 0))],
        out_shape=[jax.ShapeDtypeStruct((R, N), jnp.float32),
                   jax.ShapeDtypeStruct((8, N), jnp.float32)],
    )(x1, x2, x3, ws[0], ws[1], ws[2], bias, batch2d)


def _t8(heat_sig, pos, batch2d):
    R = N_NODES
    BR = 1000
    nR = R // BR

    def body(hr, pr, br, o, accS, accD):
        r = pl.program_id(0)

        @pl.when(r == 0)
        def _():
            accS[...] = jnp.zeros_like(accS)
            accD[...] = jnp.zeros_like(accD)

        h25 = hr[...][:, 0:25]
        p3 = pr[...]
        b = br[...]
        for i in range(4):
            mf = (b == i).astype(jnp.float32)
            hm = h25 * mf
            Sc = lax.dot_general(hm, p3, (((0,), (0,)), ((), ())),
                                 preferred_element_type=jnp.float32, precision=lax.Precision.HIGHEST)
            Dc = lax.dot_general(hm, mf, (((0,), (0,)), ((), ())),
                                 preferred_element_type=jnp.float32, precision=lax.Precision.HIGHEST)
            accS[i * 32:i * 32 + 25, 0:3] = accS[i * 32:i * 32 + 25, 0:3] + Sc
            accD[i * 32:i * 32 + 25, 0:1] = accD[i * 32:i * 32 + 25, 0:1] + Dc

        @pl.when(r == nR - 1)
        def _():
            for i in range(4):
                o[i * 32:i * 32 + 25, 0:3] = (
                    accS[i * 32:i * 32 + 25, 0:3]
                    / (accD[i * 32:i * 32 + 25, 0:1] + 1e-5))

    return pl.pallas_call(
        body, grid=(nR,),
        in_specs=[pl.BlockSpec((BR, 50), lambda r: (r, 0)),
                  pl.BlockSpec((BR, 3), lambda r: (r, 0)),
                  pl.BlockSpec((BR, 1), lambda r: (r, 0))],
        out_specs=pl.BlockSpec((128, 128), lambda r: (0, 0)),
        out_shape=jax.ShapeDtypeStruct((128, 128), jnp.float32),
        scratch_shapes=[pltpu.VMEM((128, 128), jnp.float32),
                        pltpu.VMEM((128, 128), jnp.float32)],
    )(heat_sig, pos, batch2d)


def _pad2(a, r, c):
    pr = r - a.shape[0]
    pc = c - a.shape[1]
    if pr == 0 and pc == 0:
        return a
    return jnp.pad(a, ((0, pr), (0, pc)))


def _conv(x_node, p_tpl, p_geo, p_mlp, dst_t, src_t, dst_g, src_g):
    Cin = x_node.shape[1]
    l1t, l2t = p_tpl
    l1g, l2g = p_geo
    C = l1t["W"].shape[1]
    CP = -(-C // 128) * 128
    C2 = l2t["W"].shape[1]
    C2P = -(-C2 // 128) * 128
    Cin_log = l1t["W"].shape[0] // 2

    def split(l1):
        Wa = l1["W"][:Cin_log]
        Wb = l1["W"][Cin_log:]
        return _pad2(Wa - Wb, Cin, CP), _pad2(Wb, Cin, CP)

    Wda_t, Wb_t = split(l1t)
    Wda_g, Wb_g = split(l1g)
    Wcat = jnp.concatenate([Wda_t, Wb_t, Wda_g, Wb_g], axis=1)
    zp = jnp.zeros((CP,), jnp.float32)
    b1t = zp.at[:C].set(l1t["b"])
    b1g = zp.at[:C].set(l1g["b"])
    bcat = jnp.concatenate([b1t, zp, b1g, zp]).reshape(1, 4 * CP)

    A_t, B_t, A_g, B_g = _t1(x_node, Wcat, bcat, CP)
    h_t = _sc_edge_gather_add(A_t, B_t, dst_t, src_t)
    h_g = _sc_edge_gather_add(A_g, B_g, dst_g, src_g)
    b2t = jnp.zeros((1, C2P), jnp.float32).at[0, :C2].set(l2t["b"])
    b2g = jnp.zeros((1, C2P), jnp.float32).at[0, :C2].set(l2g["b"])
    m_t = _mm([h_t], [_pad2(l2t["W"], CP, C2P)], bias=b2t, act='relu',
              BR=2000, BN=C2P)
    m_g = _mm([h_g], [_pad2(l2g["W"], CP, C2P)], bias=b2g, act='relu',
              BR=2000, BN=C2P)
    o_t = _sc_scatter_max(m_t, dst_t)
    o_g = _sc_scatter_max(m_g, dst_g)
    lm = p_mlp[0]
    Cm = lm["W"].shape[1]
    return _mm([o_t, o_g],
               [_pad2(lm["W"][:C2], C2P, Cm), _pad2(lm["W"][C2:], C2P, Cm)],
               bias=lm["b"].reshape(1, Cm), act='relu', BN=min(Cm, 1024))


def kernel(pos, x, params, tpl_edge_index, geo_edge_index, batch, batch_size):
    f32 = jnp.float32
    pos = pos.astype(f32)
    x = x.astype(f32)
    src_t = tpl_edge_index[0]
    dst_t = tpl_edge_index[1]
    src_g = geo_edge_index[0]
    dst_g = geo_edge_index[1]
    batch = (batch + (jnp.asarray(batch_size, dtype=batch.dtype) - BATCH))
    b2d = batch.reshape(N_NODES, 1).astype(jnp.int32)

    x0p = jnp.concatenate([pos, x, jnp.zeros((N_NODES, 2), f32)], axis=1)

    x1 = _conv(x0p, params["gcu1_tpl"], params["gcu1_geo"], params["gcu1_mlp"],
               dst_t, src_t, dst_g, src_g)
    x2 = _conv(x1, params["gcu2_tpl"], params["gcu2_geo"], params["gcu2_mlp"],
               dst_t, src_t, dst_g, src_g)
    x3 = _conv(x2, params["gcu3_tpl"], params["gcu3_geo"], params["gcu3_mlp"],
               dst_t, src_t, dst_g, src_g)

    Wg = params["mlp_glb"][0]["W"]
    bg = params["mlp_glb"][0]["b"].reshape(1, -1)
    x4, xglb = _t4(x1, x2, x3, [Wg[0:64], Wg[64:320], Wg[320:832]], bg, b2d)

    Wh = [params[k + "_mlp"][0]["W"] for k in ("heat", "skin", "conf")]
    bh = [params[k + "_mlp"][0]["b"] for k in ("heat", "skin", "conf")]
    Wglb_cat = jnp.concatenate([w[0:1024] for w in Wh], axis=1)
    A0 = jnp.concatenate(
        [jnp.concatenate([w[1024:1030], jnp.zeros((2, 1024), f32)], axis=0)
         for w in Wh], axis=1)
    A1 = jnp.concatenate([w[1030:1094] for w in Wh], axis=1)
    A2 = jnp.concatenate([w[1094:1350] for w in Wh], axis=1)
    A3 = jnp.concatenate([w[1350:1862] for w in Wh], axis=1)
    b1cat = jnp.concatenate(bh).reshape(1, 3072)

    Gsel = _mm([xglb], [Wglb_cat], BR=8, BN=1024)
    y1 = _mm([x0p, x1, x2, x3], [A0, A1, A2, A3], bias=b1cat, act='relu',
             sel=(b2d, Gsel), BN=1024)

    y2 = []
    for hidx, k in enumerate(("heat", "skin", "conf")):
        l2 = params[k + "_mlp"][1]
        y2.append(_mm([y1], [l2["W"]], koffs=[hidx],
                      bias=l2["b"].reshape(1, 256), act='relu', BN=256))
    heat_sig = _mm([y2[0]], [params["heat_out"]["W"]],
                   bias=params["heat_out"]["b"].reshape(1, 50), act='sigmoid',
                   BN=50)
    skin = _mm([y2[1]], [params["skin_out"]["W"]],
               bias=params["skin_out"]["b"].reshape(1, 25), BN=25)
    conf = _mm([y2[2]], [params["conf_out"]["W"]],
               bias=params["conf_out"]["b"].reshape(1, 3), act='l2norm', BN=3)

    co = _t8(heat_sig, pos, b2d)
    coords = jnp.stack([co[i * 32:i * 32 + 25, 0:3] for i in range(4)], axis=0)
    return (coords, heat_sig, skin, conf)
